# Initial kernel scaffold; baseline (speedup 1.0000x reference)
import jax
import jax.numpy as jnp
from jax.experimental import pallas as pl


def kernel(adj, feats, r, V0, a0, Wsl0, V1, a1, Wsl1):
    raise NotImplementedError("write your pallas kernel here")


# speedup vs baseline: 2.5603x; 2.5603x over previous
import functools

import jax
import jax.numpy as jnp
from jax import lax
from jax.experimental import pallas as pl
from jax.experimental.pallas import tpu as pltpu
from jax.experimental.pallas import tpu_sc as plsc

N = 10000
E = 320000
D = 128
R = 16
NB = 8

NC = 2
NS = 16
NW = NC * NS
EPW = E // NW
C = 80
NCHUNK = EPW // C
NP = 10240
RPT = NP // NS
ZR = 128


def _xw_body(a_ref, v_ref, f_ref, out_ref):
    w = jax.lax.dot_general(a_ref[...], v_ref[...],
                            (((1,), (0,)), ((), ())),
                            preferred_element_type=jnp.float32)
    f = f_ref[...]
    for rr in range(R):
        out_ref[:, rr, :] = jnp.dot(f, w[rr],
                                    preferred_element_type=jnp.float32)


def _final_body(f_ref, w_ref, p_ref, out_ref):
    acc = p_ref[0] + p_ref[1] + jnp.dot(f_ref[...], w_ref[...],
                                        preferred_element_type=jnp.float32)
    out_ref[...] = jnp.maximum(acc, 0.0)


def _sc_body(src_hbm, rel_hbm, dst_hbm, xw_hbm, out_hbm,
             src_v, rel_v, dst_v, g_v, rows_v, stage_v, agg_sh, sem):
    c = lax.axis_index("c")
    s = lax.axis_index("s")
    wid = c * NS + s

    zero16 = jnp.zeros((16,), jnp.float32)

    def zrow(i, carry):
        for j in range(D // 16):
            stage_v[i, pl.ds(j * 16, 16)] = zero16
        return carry

    lax.fori_loop(0, ZR, zrow, 0)
    for k in range(RPT // ZR):
        pltpu.sync_copy(stage_v, agg_sh.at[pl.ds(s * RPT + k * ZR, ZR)])
    plsc.subcore_barrier()

    ebase = wid * EPW

    def chunk(j, carry):
        eb = ebase + j * C
        pltpu.sync_copy(src_hbm.at[pl.ds(eb, C)], src_v)
        pltpu.sync_copy(rel_hbm.at[pl.ds(eb, C)], rel_v)
        pltpu.sync_copy(dst_hbm.at[pl.ds(eb, C)], dst_v)
        for i in range(C // 16):
            sl = pl.ds(i * 16, 16)
            g_v[sl] = src_v[sl] * R + rel_v[sl]
        pltpu.async_copy(xw_hbm.at[g_v], rows_v, sem).wait()
        pltpu.sync_copy(rows_v, agg_sh.at[dst_v], add=True)
        return carry

    lax.fori_loop(0, NCHUNK, chunk, 0)
    plsc.subcore_barrier()

    for k in range(RPT // ZR):
        base = s * RPT + k * ZR
        pltpu.sync_copy(agg_sh.at[pl.ds(base, ZR)], stage_v)
        pltpu.sync_copy(stage_v, out_hbm.at[c, pl.ds(base, ZR)])


@functools.lru_cache(maxsize=None)
def _make_sc_call():
    return pl.kernel(
        _sc_body,
        mesh=plsc.VectorSubcoreMesh(core_axis_name="c", subcore_axis_name="s"),
        out_type=jax.ShapeDtypeStruct((NC, NP, D), jnp.float32),
        scratch_types=[
            pltpu.VMEM((C,), jnp.int32),
            pltpu.VMEM((C,), jnp.int32),
            pltpu.VMEM((C,), jnp.int32),
            pltpu.VMEM((C,), jnp.int32),
            pltpu.VMEM((C, D), jnp.float32),
            pltpu.VMEM((ZR, D), jnp.float32),
            pltpu.VMEM_SHARED((NP, D), jnp.float32),
            pltpu.SemaphoreType.DMA,
        ],
    )


def kernel(adj, feats, r, V0, a0, Wsl0, V1, a1, Wsl1):
    src = adj[0]
    dst = adj[1]

    BN = 1000
    xw = pl.pallas_call(
        _xw_body,
        grid=(N // BN,),
        in_specs=[
            pl.BlockSpec((R, NB), lambda i: (0, 0)),
            pl.BlockSpec((NB, D, D), lambda i: (0, 0, 0)),
            pl.BlockSpec((BN, D), lambda i: (i, 0)),
        ],
        out_specs=pl.BlockSpec((BN, R, D), lambda i: (i, 0, 0)),
        out_shape=jax.ShapeDtypeStruct((N, R, D), jnp.float32),
    )(a1, V1, feats)

    partials = _make_sc_call()(src, r, dst, xw.reshape(N * R, D))

    out = pl.pallas_call(
        _final_body,
        grid=(N // BN,),
        in_specs=[
            pl.BlockSpec((BN, D), lambda i: (i, 0)),
            pl.BlockSpec((D, D), lambda i: (0, 0)),
            pl.BlockSpec((NC, BN, D), lambda i: (0, i, 0)),
        ],
        out_specs=pl.BlockSpec((BN, D), lambda i: (i, 0)),
        out_shape=jax.ShapeDtypeStruct((N, D), jnp.float32),
    )(feats, Wsl1, partials)
    return out


# speedup vs baseline: 5.1806x; 2.0235x over previous
import functools

import jax
import jax.numpy as jnp
from jax import lax
from jax.experimental import pallas as pl
from jax.experimental.pallas import tpu as pltpu
from jax.experimental.pallas import tpu_sc as plsc

N = 10000
E = 320000
D = 128
R = 16
NB = 8

NC = 2
NS = 16
NW = NC * NS
EPW = E // NW
C = 80
NCHUNK = EPW // C
NP = 10240
RPT = NP // NS
SST = 2000


def _xw_body(a_ref, v_ref, f_ref, out_ref):
    w = jax.lax.dot_general(a_ref[...], v_ref[...],
                            (((1,), (0,)), ((), ())),
                            preferred_element_type=jnp.float32)
    f = f_ref[...]
    for rr in range(R):
        out_ref[:, rr, :] = jnp.dot(f, w[rr],
                                    preferred_element_type=jnp.float32)


def _final_body(f_ref, w_ref, p_ref, out_ref):
    acc = p_ref[0] + p_ref[1] + jnp.dot(f_ref[...], w_ref[...],
                                        preferred_element_type=jnp.float32)
    out_ref[...] = jnp.maximum(acc, 0.0)


def _sc_body(src_hbm, rel_hbm, dst_hbm, xw_hbm, out_hbm,
             g_v, srcst_v, dst2_v, rows_a, rows_b, agg_sh, sem_a, sem_b):
    c = lax.axis_index("c")
    s = lax.axis_index("s")
    wid = c * NS + s

    zero16 = jnp.zeros((16,), jnp.float32)

    def zrow(i, carry):
        for j in range(D // 16):
            rows_a[i, pl.ds(j * 16, 16)] = zero16
        return carry

    lax.fori_loop(0, C, zrow, 0)
    for k in range(RPT // C):
        pltpu.sync_copy(rows_a, agg_sh.at[pl.ds(s * RPT + k * C, C)])

    pltpu.sync_copy(rel_hbm.at[pl.ds(wid * EPW, EPW)], g_v)
    pltpu.sync_copy(dst_hbm.at[wid], dst2_v)
    for h in range(EPW // SST):
        pltpu.sync_copy(src_hbm.at[pl.ds(wid * EPW + h * SST, SST)], srcst_v)

        def gstep(i, carry):
            sl = pl.ds(h * SST + i * 16, 16)
            g_v[sl] = srcst_v[pl.ds(i * 16, 16)] * R + g_v[sl]
            return carry

        lax.fori_loop(0, SST // 16, gstep, 0)
    plsc.subcore_barrier()

    def start(cidx, rows, sem):
        pltpu.async_copy(xw_hbm.at[g_v.at[pl.ds(cidx * C, C)]], rows, sem)

    def wait(cidx, rows, sem):
        pltpu.make_async_copy(xw_hbm.at[g_v.at[pl.ds(cidx * C, C)]], rows,
                              sem).wait()

    def scatter(cidx, rows):
        pltpu.sync_copy(rows, agg_sh.at[dst2_v.at[cidx]], add=True)

    start(0, rows_a, sem_a)

    def body(jj, carry):
        ca = 2 * jj
        cb = 2 * jj + 1
        start(cb, rows_b, sem_b)
        wait(ca, rows_a, sem_a)
        scatter(ca, rows_a)
        start(cb + 1, rows_a, sem_a)
        wait(cb, rows_b, sem_b)
        scatter(cb, rows_b)
        return carry

    lax.fori_loop(0, (NCHUNK - 1) // 2, body, 0)
    wait(NCHUNK - 1, rows_a, sem_a)
    scatter(NCHUNK - 1, rows_a)
    plsc.subcore_barrier()

    for k in range(RPT // C):
        base = s * RPT + k * C
        pltpu.sync_copy(agg_sh.at[pl.ds(base, C)], rows_a)
        pltpu.sync_copy(rows_a, out_hbm.at[c, pl.ds(base, C)])


@functools.lru_cache(maxsize=None)
def _make_sc_call():
    return pl.kernel(
        _sc_body,
        mesh=plsc.VectorSubcoreMesh(core_axis_name="c", subcore_axis_name="s"),
        out_type=jax.ShapeDtypeStruct((NC, NP, D), jnp.float32),
        scratch_types=[
            pltpu.VMEM((EPW,), jnp.int32),
            pltpu.VMEM((SST,), jnp.int32),
            pltpu.VMEM((NCHUNK, C), jnp.int32),
            pltpu.VMEM((C, D), jnp.float32),
            pltpu.VMEM((C, D), jnp.float32),
            pltpu.VMEM_SHARED((NP, D), jnp.float32),
            pltpu.SemaphoreType.DMA,
            pltpu.SemaphoreType.DMA,
        ],
    )


def kernel(adj, feats, r, V0, a0, Wsl0, V1, a1, Wsl1):
    src = adj[0]
    dst = adj[1]

    BN = 1000
    xw = pl.pallas_call(
        _xw_body,
        grid=(N // BN,),
        in_specs=[
            pl.BlockSpec((R, NB), lambda i: (0, 0)),
            pl.BlockSpec((NB, D, D), lambda i: (0, 0, 0)),
            pl.BlockSpec((BN, D), lambda i: (i, 0)),
        ],
        out_specs=pl.BlockSpec((BN, R, D), lambda i: (i, 0, 0)),
        out_shape=jax.ShapeDtypeStruct((N, R, D), jnp.float32),
    )(a1, V1, feats)

    partials = _make_sc_call()(src, r, dst.reshape(NW, NCHUNK, C),
                               xw.reshape(N * R, D))

    out = pl.pallas_call(
        _final_body,
        grid=(N // BN,),
        in_specs=[
            pl.BlockSpec((BN, D), lambda i: (i, 0)),
            pl.BlockSpec((D, D), lambda i: (0, 0)),
            pl.BlockSpec((NC, BN, D), lambda i: (0, i, 0)),
        ],
        out_specs=pl.BlockSpec((BN, D), lambda i: (i, 0)),
        out_shape=jax.ShapeDtypeStruct((N, D), jnp.float32),
    )(feats, Wsl1, partials)
    return out
